# initial kernel scaffold (unmeasured)
import jax
import jax.numpy as jnp
from jax import lax
from jax.experimental import pallas as pl
from jax.experimental.pallas import tpu as pltpu

N_DEV = 4


def _ring_all_reduce(partial):
    m, n = partial.shape
    ch = m // N_DEV

    def body(p_ref, out_ref, acc, stage, recvs,
             rs_send, rs_recv, ag_send, ag_recv, loc_sem):
        d = lax.axis_index("i")
        right = lax.rem(d + 1, N_DEV)
        left = lax.rem(d + N_DEV - 1, N_DEV)

        barrier = pltpu.get_barrier_semaphore()
        for nbr in (left, right):
            pl.semaphore_signal(barrier, inc=1, device_id=(nbr,),
                                device_id_type=pl.DeviceIdType.MESH)
        pl.semaphore_wait(barrier, 2)

        def chunk(ref, idx):
            return ref.at[pl.ds(idx * ch, ch), :]

        for s in range(N_DEV - 1):
            send_idx = lax.rem(d - s + N_DEV, N_DEV)
            recv_idx = lax.rem(d - s - 1 + N_DEV, N_DEV)
            src = chunk(p_ref, send_idx) if s == 0 else acc.at[(s - 1) % 2]
            rdma = pltpu.make_async_remote_copy(
                src_ref=src, dst_ref=recvs.at[s],
                send_sem=rs_send.at[s], recv_sem=rs_recv.at[s],
                device_id=(right,), device_id_type=pl.DeviceIdType.MESH)
            rdma.start()
            cp = pltpu.make_async_copy(chunk(p_ref, recv_idx), stage, loc_sem)
            cp.start()
            cp.wait()
            rdma.wait()
            acc[(s % 2)] = recvs[s] + stage[...]

        r_idx = lax.rem(d + 1, N_DEV)
        cp = pltpu.make_async_copy(acc.at[0], chunk(out_ref, r_idx), loc_sem)
        cp.start()
        cp.wait()

        for t in range(N_DEV - 1):
            g_idx = lax.rem(d + 1 - t + N_DEV, N_DEV)
            src = acc.at[0] if t == 0 else chunk(out_ref, g_idx)
            rdma = pltpu.make_async_remote_copy(
                src_ref=src, dst_ref=chunk(out_ref, g_idx),
                send_sem=ag_send.at[t], recv_sem=ag_recv.at[t],
                device_id=(right,), device_id_type=pl.DeviceIdType.MESH)
            rdma.start()
            rdma.wait()

    return pl.pallas_call(
        body,
        out_shape=jax.ShapeDtypeStruct((m, n), jnp.float32),
        in_specs=[pl.BlockSpec(memory_space=pltpu.ANY)],
        out_specs=pl.BlockSpec(memory_space=pltpu.ANY),
        scratch_shapes=[
            pltpu.VMEM((2, ch, n), jnp.float32),
            pltpu.VMEM((ch, n), jnp.float32),
            pltpu.VMEM((N_DEV - 1, ch, n), jnp.float32),
            pltpu.SemaphoreType.DMA((N_DEV - 1,)),
            pltpu.SemaphoreType.DMA((N_DEV - 1,)),
            pltpu.SemaphoreType.DMA((N_DEV - 1,)),
            pltpu.SemaphoreType.DMA((N_DEV - 1,)),
            pltpu.SemaphoreType.DMA,
        ],
        compiler_params=pltpu.CompilerParams(collective_id=0),
    )(partial)


def kernel(x, w_mat):
    partial = lax.dot(
        x, w_mat,
        precision=lax.Precision.HIGHEST,
        preferred_element_type=jnp.float32,
    )
    y = _ring_all_reduce(partial)
    amax = jnp.max(jnp.abs(y))
    scale = amax / 448.0
    q = (y / scale).astype(jnp.float8_e4m3fn)
    return q.astype(jnp.float32) * scale


# baseline (device time: 631726 ns/iter reference)
import jax
import jax.numpy as jnp
from jax import lax
from jax.experimental import pallas as pl
from jax.experimental.pallas import tpu as pltpu

N_DEV = 4


def _ring_all_reduce(partial):
    m, n = partial.shape
    ch = m // N_DEV

    def body(p_ref, out_ref, acc, stage, recvs,
             rs_send, rs_recv, ag_send, ag_recv, loc_sem):
        d = lax.axis_index("i")
        right = lax.rem(d + 1, N_DEV)
        left = lax.rem(d + N_DEV - 1, N_DEV)

        barrier = pltpu.get_barrier_semaphore()
        for nbr in (left, right):
            pl.semaphore_signal(barrier, inc=1, device_id=(nbr,),
                                device_id_type=pl.DeviceIdType.MESH)
        pl.semaphore_wait(barrier, 2)

        def chunk(ref, idx):
            return ref.at[pl.ds(idx * ch, ch), :]

        for s in range(N_DEV - 1):
            send_idx = lax.rem(d - s + N_DEV, N_DEV)
            recv_idx = lax.rem(d - s - 1 + N_DEV, N_DEV)
            src = chunk(p_ref, send_idx) if s == 0 else acc.at[:]
            rdma = pltpu.make_async_remote_copy(
                src_ref=src, dst_ref=recvs.at[s],
                send_sem=rs_send.at[s], recv_sem=rs_recv.at[s],
                device_id=(right,), device_id_type=pl.DeviceIdType.MESH)
            rdma.start()
            cp = pltpu.make_async_copy(chunk(p_ref, recv_idx), stage, loc_sem)
            cp.start()
            cp.wait()
            rdma.wait()
            acc[...] = recvs[s] + stage[...]

        r_idx = lax.rem(d + 1, N_DEV)
        cp = pltpu.make_async_copy(acc.at[:], chunk(out_ref, r_idx), loc_sem)
        cp.start()
        cp.wait()

        for t in range(N_DEV - 1):
            g_idx = lax.rem(d + 1 - t + N_DEV, N_DEV)
            src = acc.at[:] if t == 0 else chunk(out_ref, g_idx)
            rdma = pltpu.make_async_remote_copy(
                src_ref=src, dst_ref=chunk(out_ref, g_idx),
                send_sem=ag_send.at[t], recv_sem=ag_recv.at[t],
                device_id=(right,), device_id_type=pl.DeviceIdType.MESH)
            rdma.start()
            rdma.wait()

    return pl.pallas_call(
        body,
        out_shape=jax.ShapeDtypeStruct((m, n), jnp.float32),
        in_specs=[pl.BlockSpec(memory_space=pl.ANY)],
        out_specs=pl.BlockSpec(memory_space=pl.ANY),
        scratch_shapes=[
            pltpu.VMEM((ch, n), jnp.float32),
            pltpu.VMEM((ch, n), jnp.float32),
            pltpu.VMEM((N_DEV - 1, ch, n), jnp.float32),
            pltpu.SemaphoreType.DMA((N_DEV - 1,)),
            pltpu.SemaphoreType.DMA((N_DEV - 1,)),
            pltpu.SemaphoreType.DMA((N_DEV - 1,)),
            pltpu.SemaphoreType.DMA((N_DEV - 1,)),
            pltpu.SemaphoreType.DMA,
        ],
        compiler_params=pltpu.CompilerParams(
            collective_id=0, vmem_limit_bytes=62 * 1024 * 1024),
    )(partial)


def kernel(x, w_mat):
    partial = lax.dot(
        x, w_mat,
        precision=lax.DotAlgorithmPreset.BF16_BF16_F32_X3,
        preferred_element_type=jnp.float32,
    )
    y = _ring_all_reduce(partial)
    amax = jnp.max(jnp.abs(y))
    scale = amax / 448.0
    v = y / scale
    a = jnp.abs(v)
    bits = jax.lax.bitcast_convert_type(a, jnp.int32)
    bits = bits + 0x0007FFFF + ((bits >> 20) & 1)
    bits = bits & ~0x000FFFFF
    snapped = jax.lax.bitcast_convert_type(bits, jnp.float32)
    snapped = jnp.minimum(snapped, 448.0)
    sub = jnp.round(a * 512.0) / 512.0
    snapped = jnp.where(a < 2.0 ** -6, sub, snapped)
    return jnp.sign(v) * snapped * scale


# device time: 362277 ns/iter; 1.7438x vs baseline; 1.7438x over previous
import jax
import jax.numpy as jnp
from jax import lax
from jax.experimental import pallas as pl
from jax.experimental.pallas import tpu as pltpu

N_DEV = 4


def _ring_all_reduce(partial):
    m, n = partial.shape
    ch = m // N_DEV
    hw = n // 2

    def body(p_ref, out_ref,
             acc_r, acc_l, stage_r, stage_l, recvs_r, recvs_l,
             rs_send_r, rs_recv_r, rs_send_l, rs_recv_l,
             ag_send_r, ag_recv_r, ag_send_l, ag_recv_l,
             loc_r, loc_l):
        d = lax.axis_index("i")
        right = lax.rem(d + 1, N_DEV)
        left = lax.rem(d + N_DEV - 1, N_DEV)

        barrier = pltpu.get_barrier_semaphore()
        for nbr in (left, right):
            pl.semaphore_signal(barrier, inc=1, device_id=(nbr,),
                                device_id_type=pl.DeviceIdType.MESH)
        pl.semaphore_wait(barrier, 2)

        def chunk_r(ref, idx):
            return ref.at[pl.ds(idx * ch, ch), 0:hw]

        def chunk_l(ref, idx):
            return ref.at[pl.ds(idx * ch, ch), hw:n]

        for s in range(N_DEV - 1):
            si_r = lax.rem(d - s + N_DEV, N_DEV)
            ri_r = lax.rem(d - s - 1 + N_DEV, N_DEV)
            si_l = lax.rem(d + s, N_DEV)
            ri_l = lax.rem(d + s + 1, N_DEV)
            src_r = chunk_r(p_ref, si_r) if s == 0 else acc_r.at[:]
            src_l = chunk_l(p_ref, si_l) if s == 0 else acc_l.at[:]
            rdma_r = pltpu.make_async_remote_copy(
                src_ref=src_r, dst_ref=recvs_r.at[s],
                send_sem=rs_send_r.at[s], recv_sem=rs_recv_r.at[s],
                device_id=(right,), device_id_type=pl.DeviceIdType.MESH)
            rdma_l = pltpu.make_async_remote_copy(
                src_ref=src_l, dst_ref=recvs_l.at[s],
                send_sem=rs_send_l.at[s], recv_sem=rs_recv_l.at[s],
                device_id=(left,), device_id_type=pl.DeviceIdType.MESH)
            rdma_r.start()
            rdma_l.start()
            cp_r = pltpu.make_async_copy(chunk_r(p_ref, ri_r), stage_r, loc_r)
            cp_l = pltpu.make_async_copy(chunk_l(p_ref, ri_l), stage_l, loc_l)
            cp_r.start()
            cp_l.start()
            cp_r.wait()
            cp_l.wait()
            rdma_r.wait()
            acc_r[...] = recvs_r[s] + stage_r[...]
            rdma_l.wait()
            acc_l[...] = recvs_l[s] + stage_l[...]

        cp_r = pltpu.make_async_copy(
            acc_r.at[:], chunk_r(out_ref, lax.rem(d + 1, N_DEV)), loc_r)
        cp_l = pltpu.make_async_copy(
            acc_l.at[:], chunk_l(out_ref, lax.rem(d + N_DEV - 1, N_DEV)), loc_l)
        cp_r.start()
        cp_l.start()
        cp_r.wait()
        cp_l.wait()

        for t in range(N_DEV - 1):
            g_r = lax.rem(d + 1 - t + N_DEV, N_DEV)
            g_l = lax.rem(d + N_DEV - 1 + t, N_DEV)
            src_r = acc_r.at[:] if t == 0 else chunk_r(out_ref, g_r)
            src_l = acc_l.at[:] if t == 0 else chunk_l(out_ref, g_l)
            rdma_r = pltpu.make_async_remote_copy(
                src_ref=src_r, dst_ref=chunk_r(out_ref, g_r),
                send_sem=ag_send_r.at[t], recv_sem=ag_recv_r.at[t],
                device_id=(right,), device_id_type=pl.DeviceIdType.MESH)
            rdma_l = pltpu.make_async_remote_copy(
                src_ref=src_l, dst_ref=chunk_l(out_ref, g_l),
                send_sem=ag_send_l.at[t], recv_sem=ag_recv_l.at[t],
                device_id=(left,), device_id_type=pl.DeviceIdType.MESH)
            rdma_r.start()
            rdma_l.start()
            rdma_r.wait()
            rdma_l.wait()

    nsteps = N_DEV - 1
    return pl.pallas_call(
        body,
        out_shape=jax.ShapeDtypeStruct((m, n), jnp.float32),
        in_specs=[pl.BlockSpec(memory_space=pl.ANY)],
        out_specs=pl.BlockSpec(memory_space=pl.ANY),
        scratch_shapes=[
            pltpu.VMEM((ch, hw), jnp.float32),
            pltpu.VMEM((ch, hw), jnp.float32),
            pltpu.VMEM((ch, hw), jnp.float32),
            pltpu.VMEM((ch, hw), jnp.float32),
            pltpu.VMEM((nsteps, ch, hw), jnp.float32),
            pltpu.VMEM((nsteps, ch, hw), jnp.float32),
            pltpu.SemaphoreType.DMA((nsteps,)),
            pltpu.SemaphoreType.DMA((nsteps,)),
            pltpu.SemaphoreType.DMA((nsteps,)),
            pltpu.SemaphoreType.DMA((nsteps,)),
            pltpu.SemaphoreType.DMA((nsteps,)),
            pltpu.SemaphoreType.DMA((nsteps,)),
            pltpu.SemaphoreType.DMA((nsteps,)),
            pltpu.SemaphoreType.DMA((nsteps,)),
            pltpu.SemaphoreType.DMA,
            pltpu.SemaphoreType.DMA,
        ],
        compiler_params=pltpu.CompilerParams(
            collective_id=0, vmem_limit_bytes=62 * 1024 * 1024),
    )(partial)


def kernel(x, w_mat):
    partial = lax.dot(
        x, w_mat,
        precision=lax.DotAlgorithmPreset.BF16_BF16_F32_X3,
        preferred_element_type=jnp.float32,
    )
    y = _ring_all_reduce(partial)
    amax = jnp.max(jnp.abs(y))
    scale = amax / 448.0
    v = y / scale
    a = jnp.abs(v)
    bits = jax.lax.bitcast_convert_type(a, jnp.int32)
    bits = bits + 0x0007FFFF + ((bits >> 20) & 1)
    bits = bits & ~0x000FFFFF
    snapped = jax.lax.bitcast_convert_type(bits, jnp.float32)
    snapped = jnp.minimum(snapped, 448.0)
    sub = jnp.round(a * 512.0) / 512.0
    snapped = jnp.where(a < 2.0 ** -6, sub, snapped)
    return jnp.sign(v) * snapped * scale


# device time: 245575 ns/iter; 2.5724x vs baseline; 1.4752x over previous
import jax
import jax.numpy as jnp
from jax import lax
from jax.experimental import pallas as pl
from jax.experimental.pallas import tpu as pltpu

N_DEV = 4


def _all_reduce_quant(partial):
    m, n = partial.shape
    ch = m // N_DEV
    hw = n // 2

    def body(p_ref, out_ref,
             acc_r, acc_l, stage_r, stage_l, recvs_r, recvs_l,
             qmine_r, qmine_l, qrecv_r, qrecv_l, am_buf, am_recv,
             rs_send_r, rs_recv_r, rs_send_l, rs_recv_l,
             ag_send_r, ag_recv_r, ag_send_l, ag_recv_l,
             am_send_s, am_recv_s, loc_r, loc_l):
        d = lax.axis_index("i")
        right = lax.rem(d + 1, N_DEV)
        left = lax.rem(d + N_DEV - 1, N_DEV)

        barrier = pltpu.get_barrier_semaphore()
        for nbr in (left, right):
            pl.semaphore_signal(barrier, inc=1, device_id=(nbr,),
                                device_id_type=pl.DeviceIdType.MESH)
        pl.semaphore_wait(barrier, 2)

        def chunk_r(ref, idx):
            return ref.at[pl.ds(idx * ch, ch), 0:hw]

        def chunk_l(ref, idx):
            return ref.at[pl.ds(idx * ch, ch), hw:n]

        for s in range(N_DEV - 1):
            si_r = lax.rem(d - s + N_DEV, N_DEV)
            ri_r = lax.rem(d - s - 1 + N_DEV, N_DEV)
            si_l = lax.rem(d + s, N_DEV)
            ri_l = lax.rem(d + s + 1, N_DEV)
            src_r = chunk_r(p_ref, si_r) if s == 0 else acc_r.at[:]
            src_l = chunk_l(p_ref, si_l) if s == 0 else acc_l.at[:]
            rdma_r = pltpu.make_async_remote_copy(
                src_ref=src_r, dst_ref=recvs_r.at[s],
                send_sem=rs_send_r.at[s], recv_sem=rs_recv_r.at[s],
                device_id=(right,), device_id_type=pl.DeviceIdType.MESH)
            rdma_l = pltpu.make_async_remote_copy(
                src_ref=src_l, dst_ref=recvs_l.at[s],
                send_sem=rs_send_l.at[s], recv_sem=rs_recv_l.at[s],
                device_id=(left,), device_id_type=pl.DeviceIdType.MESH)
            rdma_r.start()
            rdma_l.start()
            cp_r = pltpu.make_async_copy(chunk_r(p_ref, ri_r), stage_r, loc_r)
            cp_l = pltpu.make_async_copy(chunk_l(p_ref, ri_l), stage_l, loc_l)
            cp_r.start()
            cp_l.start()
            cp_r.wait()
            cp_l.wait()
            rdma_r.wait()
            acc_r[...] = recvs_r[s] + stage_r[...]
            rdma_l.wait()
            acc_l[...] = recvs_l[s] + stage_l[...]


        m_loc = jnp.maximum(jnp.max(jnp.abs(acc_r[...])),
                            jnp.max(jnp.abs(acc_l[...])))
        am_buf[...] = jnp.full((8, 128), m_loc, jnp.float32)
        for h in range(N_DEV - 1):
            am = pltpu.make_async_remote_copy(
                src_ref=am_buf.at[:], dst_ref=am_recv.at[h],
                send_sem=am_send_s.at[h], recv_sem=am_recv_s.at[h],
                device_id=(right,), device_id_type=pl.DeviceIdType.MESH)
            am.start()
            am.wait()
            am_buf[...] = jnp.maximum(am_buf[...], am_recv[h])
        scale = jnp.max(am_buf[...]) / 448.0

        qmine_r[...] = jnp.clip(acc_r[...] / scale, -448.0, 448.0
                                ).astype(jnp.float8_e4m3fn)
        qmine_l[...] = jnp.clip(acc_l[...] / scale, -448.0, 448.0
                                ).astype(jnp.float8_e4m3fn)

        pending = []
        for t in range(N_DEV - 1):
            src_r = qmine_r.at[:] if t == 0 else qrecv_r.at[t - 1]
            src_l = qmine_l.at[:] if t == 0 else qrecv_l.at[t - 1]
            rdma_r = pltpu.make_async_remote_copy(
                src_ref=src_r, dst_ref=qrecv_r.at[t],
                send_sem=ag_send_r.at[t], recv_sem=ag_recv_r.at[t],
                device_id=(right,), device_id_type=pl.DeviceIdType.MESH)
            rdma_l = pltpu.make_async_remote_copy(
                src_ref=src_l, dst_ref=qrecv_l.at[t],
                send_sem=ag_send_l.at[t], recv_sem=ag_recv_l.at[t],
                device_id=(left,), device_id_type=pl.DeviceIdType.MESH)
            rdma_r.start()
            rdma_l.start()
            pending.append((rdma_r, rdma_l))
            if t == 0:
                q_r, q_l = qmine_r, qmine_l
                row_r = lax.rem(d + 1, N_DEV)
                row_l = lax.rem(d + N_DEV - 1, N_DEV)
            else:
                q_r, q_l = qrecv_r.at[t - 1], qrecv_l.at[t - 1]
                row_r = lax.rem(d - (t - 1) + N_DEV, N_DEV)
                row_l = lax.rem(d + (t - 1), N_DEV)
            stage_r[...] = q_r[...].astype(jnp.float32) * scale
            cp_r = pltpu.make_async_copy(stage_r, chunk_r(out_ref, row_r), loc_r)
            cp_r.start()
            stage_l[...] = q_l[...].astype(jnp.float32) * scale
            cp_l = pltpu.make_async_copy(stage_l, chunk_l(out_ref, row_l), loc_l)
            cp_l.start()
            cp_r.wait()
            cp_l.wait()
            rdma_r.wait_recv()
            rdma_l.wait_recv()
        stage_r[...] = qrecv_r[N_DEV - 2].astype(jnp.float32) * scale
        cp_r = pltpu.make_async_copy(
            stage_r, chunk_r(out_ref, lax.rem(d - (N_DEV - 2) + N_DEV, N_DEV)),
            loc_r)
        cp_r.start()
        stage_l[...] = qrecv_l[N_DEV - 2].astype(jnp.float32) * scale
        cp_l = pltpu.make_async_copy(
            stage_l, chunk_l(out_ref, lax.rem(d + N_DEV - 2, N_DEV)), loc_l)
        cp_l.start()
        cp_r.wait()
        cp_l.wait()
        for rdma_r, rdma_l in pending:
            rdma_r.wait_send()
            rdma_l.wait_send()

    nsteps = N_DEV - 1
    f8 = jnp.float8_e4m3fn
    return pl.pallas_call(
        body,
        out_shape=jax.ShapeDtypeStruct((m, n), jnp.float32),
        in_specs=[pl.BlockSpec(memory_space=pl.ANY)],
        out_specs=pl.BlockSpec(memory_space=pl.ANY),
        scratch_shapes=[
            pltpu.VMEM((ch, hw), jnp.float32),
            pltpu.VMEM((ch, hw), jnp.float32),
            pltpu.VMEM((ch, hw), jnp.float32),
            pltpu.VMEM((ch, hw), jnp.float32),
            pltpu.VMEM((nsteps, ch, hw), jnp.float32),
            pltpu.VMEM((nsteps, ch, hw), jnp.float32),
            pltpu.VMEM((ch, hw), f8),
            pltpu.VMEM((ch, hw), f8),
            pltpu.VMEM((nsteps, ch, hw), f8),
            pltpu.VMEM((nsteps, ch, hw), f8),
            pltpu.VMEM((8, 128), jnp.float32),
            pltpu.VMEM((nsteps, 8, 128), jnp.float32),
            pltpu.SemaphoreType.DMA((nsteps,)),
            pltpu.SemaphoreType.DMA((nsteps,)),
            pltpu.SemaphoreType.DMA((nsteps,)),
            pltpu.SemaphoreType.DMA((nsteps,)),
            pltpu.SemaphoreType.DMA((nsteps,)),
            pltpu.SemaphoreType.DMA((nsteps,)),
            pltpu.SemaphoreType.DMA((nsteps,)),
            pltpu.SemaphoreType.DMA((nsteps,)),
            pltpu.SemaphoreType.DMA((nsteps,)),
            pltpu.SemaphoreType.DMA((nsteps,)),
            pltpu.SemaphoreType.DMA,
            pltpu.SemaphoreType.DMA,
        ],
        compiler_params=pltpu.CompilerParams(
            collective_id=0, vmem_limit_bytes=62 * 1024 * 1024),
    )(partial)


def kernel(x, w_mat):
    partial = lax.dot(
        x, w_mat,
        precision=lax.DotAlgorithmPreset.BF16_BF16_F32_X3,
        preferred_element_type=jnp.float32,
    )
    return _all_reduce_quant(partial)
